# baseline (speedup 1.0000x reference)
import functools

import jax
import jax.numpy as jnp
from jax import lax
from jax.experimental import pallas as pl
from jax.experimental.pallas import tpu as pltpu
from jax.experimental.pallas import tpu_sc as plsc

B = 16384
D = 64
NC = 2
NS = 16
NW = NC * NS
BPW = B // NW
CH = 128
NCH = BPW // CH
L = 16


def _mf_body(user_hbm, item_hbm, ut_hbm, it_hbm, out_hbm,
             uidx, iidx, urows, irows, pads, fin, outv, sem):
    wid = lax.axis_index("s") * NC + lax.axis_index("c")
    base = wid * BPW

    pltpu.sync_copy(user_hbm.at[pl.ds(base, BPW)], uidx)
    pltpu.sync_copy(item_hbm.at[pl.ds(base, BPW)], iidx)

    copies = []
    for k in range(NCH):
        copies.append(pltpu.async_copy(
            ut_hbm.at[uidx.at[pl.ds(k * CH, CH)]],
            urows.at[pl.ds(k * CH, CH)], sem))
        copies.append(pltpu.async_copy(
            it_hbm.at[iidx.at[pl.ds(k * CH, CH)]],
            irows.at[pl.ds(k * CH, CH)], sem))
    for c in copies:
        c.wait()

    zero = jnp.zeros((L,), jnp.float32)
    for r in range(L):
        pads[r, pl.ds(L, L)] = zero

    def chunk_body(c, carry):
        row0 = c * L
        for r in range(L):
            row = row0 + r
            p = None
            for j in range(D // L):
                u = urows[row, pl.ds(j * L, L)]
                v = irows[row, pl.ds(j * L, L)]
                prod = u * v
                p = prod if p is None else p + prod
            y = p
            for off in (8, 4, 2, 1):
                pads[r, pl.ds(0, L)] = y
                y = y + pads[r, pl.ds(off, L)]
            fin[pl.ds(r, L)] = y
        q = fin[pl.ds(0, L)]
        outv[pl.ds(row0, L)] = 1.0 / (1.0 + jnp.exp(-q))
        return carry

    lax.fori_loop(0, BPW // L, chunk_body, 0)

    pltpu.sync_copy(outv, out_hbm.at[pl.ds(base, BPW)])


def kernel(user, item, user_table, item_table):
    mesh = plsc.VectorSubcoreMesh(core_axis_name="c", subcore_axis_name="s")
    run = functools.partial(
        pl.kernel,
        out_type=jax.ShapeDtypeStruct((B,), jnp.float32),
        mesh=mesh,
        compiler_params=pltpu.CompilerParams(use_tc_tiling_on_sc=False),
        scratch_types=[
            pltpu.VMEM((BPW,), jnp.int32),
            pltpu.VMEM((BPW,), jnp.int32),
            pltpu.VMEM((BPW, D), jnp.float32),
            pltpu.VMEM((BPW, D), jnp.float32),
            pltpu.VMEM((L, 2 * L), jnp.float32),
            pltpu.VMEM((2 * L,), jnp.float32),
            pltpu.VMEM((BPW,), jnp.float32),
            pltpu.SemaphoreType.DMA,
        ],
    )(_mf_body)
    return run(user, item, user_table, item_table)


# speedup vs baseline: 2.3866x; 2.3866x over previous
import functools

import jax
import jax.numpy as jnp
from jax import lax
from jax.experimental import pallas as pl
from jax.experimental.pallas import tpu as pltpu
from jax.experimental.pallas import tpu_sc as plsc

B = 16384
D = 64
NC = 2
NS = 16
NW = NC * NS
BPW = B // NW
L = 16
WAVES = BPW // 2


def _mf_body(user_hbm, item_hbm, ut_hbm, it_hbm, out_hbm,
             uidx, iidx,
             ub00, ub01, ub10, ub11, vb00, vb01, vb10, vb11,
             outv, sem0, sem1):
    wid = lax.axis_index("s") * NC + lax.axis_index("c")
    base = wid * BPW

    ub = ((ub00, ub01), (ub10, ub11))
    vb = ((vb00, vb01), (vb10, vb11))
    sems = (sem0, sem1)

    pltpu.sync_copy(user_hbm.at[pl.ds(base, BPW)], uidx.at[pl.ds(0, BPW)])
    pltpu.sync_copy(item_hbm.at[pl.ds(base, BPW)], iidx.at[pl.ds(0, BPW)])

    lanes = lax.iota(jnp.int32, L)
    zeros = jnp.zeros((L,), jnp.float32)

    def fire(w, parity):
        i0 = jnp.minimum(w * 2, BPW - 2)
        uv = uidx[pl.ds(i0, L)]
        iv = iidx[pl.ds(i0, L)]
        for k in range(2):
            cu = pl.multiple_of((uv[k] >> 7) * 128, 128)
            cv = pl.multiple_of((iv[k] >> 7) * 128, 128)
            pltpu.async_copy(ut_hbm.at[:, pl.ds(cu, 128)], ub[parity][k],
                             sems[parity])
            pltpu.async_copy(it_hbm.at[:, pl.ds(cv, 128)], vb[parity][k],
                             sems[parity])

    fire(0, 0)

    def half_wave(w, q, p):
        fire(w + 1, 1 - p)
        for _ in range(4):
            pltpu.make_async_copy(ut_hbm.at[:, pl.ds(0, 128)],
                                  ub[0][0], sems[p]).wait()
        i0 = w * 2
        uv = uidx[pl.ds(i0, L)]
        iv = iidx[pl.ds(i0, L)]
        for k in range(2):
            lu = jnp.full((L,), uv[k] & 127, jnp.int32)
            lv = jnp.full((L,), iv[k] & 127, jnp.int32)
            acc = None
            for j in range(D // L):
                rows = lanes + (j * L)
                uc = plsc.load_gather(ub[p][k], [rows, lu])
                vc = plsc.load_gather(vb[p][k], [rows, lv])
                prod = uc * vc
                acc = prod if acc is None else acc + prod
            s = jnp.sum(acc)
            q = jnp.where(lanes == ((i0 + k) & 15), s, q)
        return q

    def pair_body(t, q):
        q = half_wave(2 * t, q, 0)
        w1 = 2 * t + 1
        q = half_wave(w1, q, 1)
        flush = (w1 & 7) == 7
        @pl.when(flush)
        def _():
            outv[pl.ds((w1 >> 3) * L, L)] = 1.0 / (1.0 + jnp.exp(-q))
        return jnp.where(flush, zeros, q)

    lax.fori_loop(0, WAVES // 2, pair_body, zeros)

    for _ in range(4):
        pltpu.make_async_copy(ut_hbm.at[:, pl.ds(0, 128)],
                              ub[0][0], sems[WAVES & 1]).wait()

    pltpu.sync_copy(outv, out_hbm.at[pl.ds(base, BPW)])


def kernel(user, item, user_table, item_table):
    mesh = plsc.VectorSubcoreMesh(core_axis_name="c", subcore_axis_name="s")
    blk = lambda: pltpu.VMEM((D, 128), jnp.float32)
    run = functools.partial(
        pl.kernel,
        out_type=jax.ShapeDtypeStruct((B,), jnp.float32),
        mesh=mesh,
        compiler_params=pltpu.CompilerParams(needs_layout_passes=False),
        scratch_types=[
            pltpu.VMEM((BPW + L,), jnp.int32),
            pltpu.VMEM((BPW + L,), jnp.int32),
            blk(), blk(), blk(), blk(),
            blk(), blk(), blk(), blk(),
            pltpu.VMEM((BPW,), jnp.float32),
            pltpu.SemaphoreType.DMA,
            pltpu.SemaphoreType.DMA,
        ],
    )(_mf_body)
    return run(user, item, user_table.T, item_table.T)


# speedup vs baseline: 2.6259x; 1.1003x over previous
import functools

import jax
import jax.numpy as jnp
from jax import lax
from jax.experimental import pallas as pl
from jax.experimental.pallas import tpu as pltpu
from jax.experimental.pallas import tpu_sc as plsc

B = 16384
D = 64
NC = 2
NS = 16
NW = NC * NS
BPW = B // NW
L = 16
WAVES = BPW // 2


DEPTH = 4


def _mf_body(user_hbm, item_hbm, ut_hbm, it_hbm, out_hbm,
             uidx, iidx,
             ub0, ub1, ub2, ub3, vb0, vb1, vb2, vb3,
             outv, sem0, sem1, sem2, sem3):
    wid = lax.axis_index("s") * NC + lax.axis_index("c")
    base = wid * BPW

    ub = (ub0, ub1, ub2, ub3)
    vb = (vb0, vb1, vb2, vb3)
    sems = (sem0, sem1, sem2, sem3)

    pltpu.sync_copy(user_hbm.at[pl.ds(base, BPW)], uidx.at[pl.ds(0, BPW)])
    pltpu.sync_copy(item_hbm.at[pl.ds(base, BPW)], iidx.at[pl.ds(0, BPW)])

    lanes = lax.iota(jnp.int32, L)
    zeros = jnp.zeros((L,), jnp.float32)

    def fire(w, parity):
        i0 = jnp.minimum(w, BPW - 1)
        uv = uidx[pl.ds(i0, L)]
        iv = iidx[pl.ds(i0, L)]
        cu = pl.multiple_of((uv[0] >> 7) * 128, 128)
        cv = pl.multiple_of((iv[0] >> 7) * 128, 128)
        pltpu.async_copy(ut_hbm.at[:, pl.ds(cu, 128)], ub[parity],
                         sems[parity])
        pltpu.async_copy(it_hbm.at[:, pl.ds(cv, 128)], vb[parity],
                         sems[parity])

    for s in range(DEPTH):
        fire(s, s)

    def quad_body(t, q):
        for s in range(DEPTH):
            w = DEPTH * t + s
            for _ in range(2):
                pltpu.make_async_copy(ut_hbm.at[:, pl.ds(0, 128)],
                                      ub[0], sems[s]).wait()
            uv = uidx[pl.ds(w, L)]
            iv = iidx[pl.ds(w, L)]
            lu = jnp.full((L,), uv[0] & 127, jnp.int32)
            lv = jnp.full((L,), iv[0] & 127, jnp.int32)
            acc = None
            for j in range(D // L):
                rows = lanes + (j * L)
                uc = plsc.load_gather(ub[s], [rows, lu])
                vc = plsc.load_gather(vb[s], [rows, lv])
                prod = uc * vc
                acc = prod if acc is None else acc + prod
            q = jnp.where(lanes == (w & 15), jnp.sum(acc), q)
            fire(w + DEPTH, s)
            if s == DEPTH - 1:
                flush = (w & 15) == 15
                @pl.when(flush)
                def _():
                    outv[pl.ds((w >> 4) * L, L)] = 1.0 / (1.0 + jnp.exp(-q))
                q = jnp.where(flush, zeros, q)
        return q

    lax.fori_loop(0, BPW // DEPTH, quad_body, zeros)

    for s in range(DEPTH):
        for _ in range(2):
            pltpu.make_async_copy(ut_hbm.at[:, pl.ds(0, 128)],
                                  ub[0], sems[s]).wait()

    pltpu.sync_copy(outv, out_hbm.at[pl.ds(base, BPW)])


def kernel(user, item, user_table, item_table):
    mesh = plsc.VectorSubcoreMesh(core_axis_name="c", subcore_axis_name="s")
    blk = lambda: pltpu.VMEM((D, 128), jnp.float32)
    run = functools.partial(
        pl.kernel,
        out_type=jax.ShapeDtypeStruct((B,), jnp.float32),
        mesh=mesh,
        compiler_params=pltpu.CompilerParams(needs_layout_passes=False),
        scratch_types=[
            pltpu.VMEM((BPW + L,), jnp.int32),
            pltpu.VMEM((BPW + L,), jnp.int32),
            blk(), blk(), blk(), blk(),
            blk(), blk(), blk(), blk(),
            pltpu.VMEM((BPW,), jnp.float32),
            pltpu.SemaphoreType.DMA,
            pltpu.SemaphoreType.DMA,
            pltpu.SemaphoreType.DMA,
            pltpu.SemaphoreType.DMA,
        ],
    )(_mf_body)
    return run(user, item, user_table.T, item_table.T)
